# baseline (speedup 1.0000x reference)
import jax
import jax.numpy as jnp
from jax import lax
from jax.experimental import pallas as pl
from jax.experimental.pallas import tpu as pltpu
from jax.experimental.pallas import tpu_sc as plsc

ROWS = 8192
COLS = 4096
NC = 2
NS = 16
NW = NC * NS
RPW = ROWS // NW
RB = 8
NBLK = RPW // RB
CW = 512
NCHUNK = COLS // CW
VPC = CW // 16
NPOS = NBLK * NCHUNK

_BCAST15_DNUMS = lax.GatherDimensionNumbers(
    offset_dims=(), collapsed_slice_dims=(0,), start_index_map=(0,)
)


def _bcast_last(v):
    idx = jnp.full((16, 1), 15, jnp.int32)
    return lax.gather(v, idx, _BCAST15_DNUMS, slice_sizes=(1,),
                      mode=lax.GatherScatterMode.PROMISE_IN_BOUNDS)


def _body(x_hbm, out_hbm, in_a, in_b, out_a, out_b,
          isem_a, isem_b, osem_a, osem_b):
    c = lax.axis_index("c")
    s = lax.axis_index("s")
    wid = c * NS + s
    row0 = wid * RPW

    def start_in(p, buf, sem):
        blk = p // NCHUNK
        ch = p % NCHUNK
        pltpu.async_copy(
            x_hbm.at[pl.ds(row0 + RB * blk, RB), pl.ds(CW * ch, CW)],
            buf, sem)

    def wait_in(buf, sem):
        pltpu.make_async_copy(
            x_hbm.at[pl.ds(0, RB), pl.ds(0, CW)], buf, sem).wait()

    def start_out(p, buf, sem):
        blk = p // NCHUNK
        ch = p % NCHUNK
        pltpu.async_copy(
            buf,
            out_hbm.at[pl.ds(row0 + RB * blk, RB), pl.ds(CW * ch, CW)],
            sem)

    def wait_out(buf, sem):
        pltpu.make_async_copy(
            buf, out_hbm.at[pl.ds(0, RB), pl.ds(0, CW)], sem).wait()

    def compute(p, in_buf, out_buf, carry):
        fresh = (p % NCHUNK) == 0
        carry = tuple(
            jnp.where(fresh, jnp.zeros((16,), jnp.float32), carry[r])
            for r in range(RB)
        )

        @plsc.parallel_loop(0, VPC, step=1, unroll=4, carry=carry)
        def vstep(v, carry):
            new = []
            for r in range(RB):
                x = in_buf[r, pl.ds(16 * v, 16)]
                sc = plsc.cumsum(x)
                out_buf[r, pl.ds(16 * v, 16)] = sc + carry[r]
                new.append(carry[r] + _bcast_last(sc))
            return tuple(new)

        return vstep

    carry0 = tuple(jnp.zeros((16,), jnp.float32) for _ in range(RB))

    start_in(0, in_a, isem_a)
    start_in(1, in_b, isem_b)

    wait_in(in_a, isem_a)
    carry = compute(0, in_a, out_a, carry0)
    start_out(0, out_a, osem_a)
    start_in(2, in_a, isem_a)

    wait_in(in_b, isem_b)
    carry = compute(1, in_b, out_b, carry)
    start_out(1, out_b, osem_b)
    start_in(3, in_b, isem_b)

    def pair_step(t, carry):
        pa = 2 * t
        pb = 2 * t + 1

        wait_in(in_a, isem_a)
        wait_out(out_a, osem_a)
        carry = compute(pa, in_a, out_a, carry)
        start_out(pa, out_a, osem_a)
        start_in(jnp.minimum(pa + 2, NPOS - 1), in_a, isem_a)

        wait_in(in_b, isem_b)
        wait_out(out_b, osem_b)
        carry = compute(pb, in_b, out_b, carry)
        start_out(pb, out_b, osem_b)
        start_in(jnp.minimum(pb + 2, NPOS - 1), in_b, isem_b)
        return carry

    lax.fori_loop(1, NPOS // 2, pair_step, carry)

    wait_in(in_a, isem_a)
    wait_in(in_b, isem_b)
    wait_out(out_a, osem_a)
    wait_out(out_b, osem_b)


def kernel(x):
    mesh = plsc.VectorSubcoreMesh(core_axis_name="c", subcore_axis_name="s")
    run = pl.kernel(
        _body,
        out_type=jax.ShapeDtypeStruct((ROWS, COLS), jnp.float32),
        mesh=mesh,
        scratch_types=[
            pltpu.VMEM((RB, CW), jnp.float32),
            pltpu.VMEM((RB, CW), jnp.float32),
            pltpu.VMEM((RB, CW), jnp.float32),
            pltpu.VMEM((RB, CW), jnp.float32),
            pltpu.SemaphoreType.DMA,
            pltpu.SemaphoreType.DMA,
            pltpu.SemaphoreType.DMA,
            pltpu.SemaphoreType.DMA,
        ],
        compiler_params=pltpu.CompilerParams(
            use_tc_tiling_on_sc=True, needs_layout_passes=False
        ),
    )
    return run(x)


# speedup vs baseline: 1.3209x; 1.3209x over previous
import jax
import jax.numpy as jnp
from jax import lax
from jax.experimental import pallas as pl
from jax.experimental.pallas import tpu as pltpu
from jax.experimental.pallas import tpu_sc as plsc

ROWS = 8192
COLS = 4096
NC = 2
NS = 16
NW = NC * NS
RPW = ROWS // NW
RB = 8
NBLK = RPW // RB
CW = 2048
NCHUNK = COLS // CW
VPC = CW // 16
NPOS = NBLK * NCHUNK

_BCAST15_DNUMS = lax.GatherDimensionNumbers(
    offset_dims=(), collapsed_slice_dims=(0,), start_index_map=(0,)
)


def _bcast_last(v):
    idx = jnp.full((16, 1), 15, jnp.int32)
    return lax.gather(v, idx, _BCAST15_DNUMS, slice_sizes=(1,),
                      mode=lax.GatherScatterMode.PROMISE_IN_BOUNDS)


def _body(x_hbm, out_hbm, in_a, in_b, out_a, out_b,
          isem_a, isem_b, osem_a, osem_b):
    c = lax.axis_index("c")
    s = lax.axis_index("s")
    wid = c * NS + s
    row0 = wid * RPW

    def start_in(p, buf, sem):
        blk = p // NCHUNK
        ch = p % NCHUNK
        pltpu.async_copy(
            x_hbm.at[pl.ds(row0 + RB * blk, RB), pl.ds(CW * ch, CW)],
            buf, sem)

    def wait_in(buf, sem):
        pltpu.make_async_copy(
            x_hbm.at[pl.ds(0, RB), pl.ds(0, CW)], buf, sem).wait()

    def start_out(p, buf, sem):
        blk = p // NCHUNK
        ch = p % NCHUNK
        pltpu.async_copy(
            buf,
            out_hbm.at[pl.ds(row0 + RB * blk, RB), pl.ds(CW * ch, CW)],
            sem)

    def wait_out(buf, sem):
        pltpu.make_async_copy(
            buf, out_hbm.at[pl.ds(0, RB), pl.ds(0, CW)], sem).wait()

    def compute(p, in_buf, out_buf, carry):
        fresh = (p % NCHUNK) == 0
        carry = tuple(
            jnp.where(fresh, jnp.zeros((16,), jnp.float32), carry[r])
            for r in range(RB)
        )

        @plsc.parallel_loop(0, VPC, step=1, unroll=4, carry=carry)
        def vstep(v, carry):
            new = []
            for r in range(RB):
                x = in_buf[r, pl.ds(16 * v, 16)]
                sc = plsc.cumsum(x)
                out_buf[r, pl.ds(16 * v, 16)] = sc + carry[r]
                new.append(carry[r] + _bcast_last(sc))
            return tuple(new)

        return vstep

    carry0 = tuple(jnp.zeros((16,), jnp.float32) for _ in range(RB))

    start_in(0, in_a, isem_a)
    start_in(1, in_b, isem_b)

    wait_in(in_a, isem_a)
    carry = compute(0, in_a, out_a, carry0)
    start_out(0, out_a, osem_a)
    start_in(2, in_a, isem_a)

    wait_in(in_b, isem_b)
    carry = compute(1, in_b, out_b, carry)
    start_out(1, out_b, osem_b)
    start_in(3, in_b, isem_b)

    def pair_step(t, carry):
        pa = 2 * t
        pb = 2 * t + 1

        wait_in(in_a, isem_a)
        wait_out(out_a, osem_a)
        carry = compute(pa, in_a, out_a, carry)
        start_out(pa, out_a, osem_a)
        start_in(jnp.minimum(pa + 2, NPOS - 1), in_a, isem_a)

        wait_in(in_b, isem_b)
        wait_out(out_b, osem_b)
        carry = compute(pb, in_b, out_b, carry)
        start_out(pb, out_b, osem_b)
        start_in(jnp.minimum(pb + 2, NPOS - 1), in_b, isem_b)
        return carry

    lax.fori_loop(1, NPOS // 2, pair_step, carry)

    wait_in(in_a, isem_a)
    wait_in(in_b, isem_b)
    wait_out(out_a, osem_a)
    wait_out(out_b, osem_b)


def kernel(x):
    mesh = plsc.VectorSubcoreMesh(core_axis_name="c", subcore_axis_name="s")
    run = pl.kernel(
        _body,
        out_type=jax.ShapeDtypeStruct((ROWS, COLS), jnp.float32),
        mesh=mesh,
        scratch_types=[
            pltpu.VMEM((RB, CW), jnp.float32),
            pltpu.VMEM((RB, CW), jnp.float32),
            pltpu.VMEM((RB, CW), jnp.float32),
            pltpu.VMEM((RB, CW), jnp.float32),
            pltpu.SemaphoreType.DMA,
            pltpu.SemaphoreType.DMA,
            pltpu.SemaphoreType.DMA,
            pltpu.SemaphoreType.DMA,
        ],
        compiler_params=pltpu.CompilerParams(
            use_tc_tiling_on_sc=True, needs_layout_passes=False
        ),
    )
    return run(x)


# speedup vs baseline: 1.3968x; 1.0574x over previous
import jax
import jax.numpy as jnp
from jax import lax
from jax.experimental import pallas as pl
from jax.experimental.pallas import tpu as pltpu
from jax.experimental.pallas import tpu_sc as plsc

ROWS = 8192
COLS = 4096
NC = 2
NS = 16
NW = NC * NS
RPW = ROWS // NW
RB = 8
NBLK = RPW // RB
CW = 2048
NCHUNK = COLS // CW
VPC = CW // 16
NPOS = NBLK * NCHUNK

_BCAST15_DNUMS = lax.GatherDimensionNumbers(
    offset_dims=(), collapsed_slice_dims=(0,), start_index_map=(0,)
)


def _bcast_last(v):
    idx = jnp.full((16, 1), 15, jnp.int32)
    return lax.gather(v, idx, _BCAST15_DNUMS, slice_sizes=(1,),
                      mode=lax.GatherScatterMode.PROMISE_IN_BOUNDS)


def _body(x_hbm, out_hbm, in_a, in_b, out_a, out_b,
          isem_a, isem_b, osem_a, osem_b):
    c = lax.axis_index("c")
    s = lax.axis_index("s")
    wid = c * NS + s
    row0 = wid * RPW

    def start_in(p, buf, sem):
        blk = p // NCHUNK
        ch = p % NCHUNK
        pltpu.async_copy(
            x_hbm.at[pl.ds(row0 + RB * blk, RB), pl.ds(CW * ch, CW)],
            buf, sem)

    def wait_in(buf, sem):
        pltpu.make_async_copy(
            x_hbm.at[pl.ds(0, RB), pl.ds(0, CW)], buf, sem).wait()

    def start_out(p, buf, sem):
        blk = p // NCHUNK
        ch = p % NCHUNK
        pltpu.async_copy(
            buf,
            out_hbm.at[pl.ds(row0 + RB * blk, RB), pl.ds(CW * ch, CW)],
            sem)

    def wait_out(buf, sem):
        pltpu.make_async_copy(
            buf, out_hbm.at[pl.ds(0, RB), pl.ds(0, CW)], sem).wait()

    def compute(p, in_buf, out_buf, carry):
        fresh = (p % NCHUNK) == 0
        carry = tuple(
            jnp.where(fresh, jnp.zeros((16,), jnp.float32), carry[r])
            for r in range(RB)
        )

        @plsc.parallel_loop(0, VPC, step=1, unroll=4, carry=carry)
        def vstep(v, carry):
            new = []
            for r in range(RB):
                x = in_buf[r, pl.ds(16 * v, 16)]
                out = plsc.cumsum(x) + carry[r]
                out_buf[r, pl.ds(16 * v, 16)] = out
                new.append(_bcast_last(out))
            return tuple(new)

        return vstep

    carry0 = tuple(jnp.zeros((16,), jnp.float32) for _ in range(RB))

    start_in(0, in_a, isem_a)
    start_in(1, in_b, isem_b)

    wait_in(in_a, isem_a)
    carry = compute(0, in_a, out_a, carry0)
    start_out(0, out_a, osem_a)
    start_in(2, in_a, isem_a)

    wait_in(in_b, isem_b)
    carry = compute(1, in_b, out_b, carry)
    start_out(1, out_b, osem_b)
    start_in(3, in_b, isem_b)

    def pair_step(t, carry):
        pa = 2 * t
        pb = 2 * t + 1

        wait_in(in_a, isem_a)
        wait_out(out_a, osem_a)
        carry = compute(pa, in_a, out_a, carry)
        start_out(pa, out_a, osem_a)
        start_in(jnp.minimum(pa + 2, NPOS - 1), in_a, isem_a)

        wait_in(in_b, isem_b)
        wait_out(out_b, osem_b)
        carry = compute(pb, in_b, out_b, carry)
        start_out(pb, out_b, osem_b)
        start_in(jnp.minimum(pb + 2, NPOS - 1), in_b, isem_b)
        return carry

    lax.fori_loop(1, NPOS // 2, pair_step, carry)

    wait_in(in_a, isem_a)
    wait_in(in_b, isem_b)
    wait_out(out_a, osem_a)
    wait_out(out_b, osem_b)


def kernel(x):
    mesh = plsc.VectorSubcoreMesh(core_axis_name="c", subcore_axis_name="s")
    run = pl.kernel(
        _body,
        out_type=jax.ShapeDtypeStruct((ROWS, COLS), jnp.float32),
        mesh=mesh,
        scratch_types=[
            pltpu.VMEM((RB, CW), jnp.float32),
            pltpu.VMEM((RB, CW), jnp.float32),
            pltpu.VMEM((RB, CW), jnp.float32),
            pltpu.VMEM((RB, CW), jnp.float32),
            pltpu.SemaphoreType.DMA,
            pltpu.SemaphoreType.DMA,
            pltpu.SemaphoreType.DMA,
            pltpu.SemaphoreType.DMA,
        ],
        compiler_params=pltpu.CompilerParams(
            use_tc_tiling_on_sc=True, needs_layout_passes=False
        ),
    )
    return run(x)
